# baseline (speedup 1.0000x reference)
import functools

import jax
import jax.numpy as jnp
from jax import lax
from jax.experimental import pallas as pl
from jax.experimental.pallas import tpu as pltpu
from jax.experimental.pallas import tpu_sc as plsc

_BATCH = 16384
_DIM = 2048

@functools.lru_cache(maxsize=1)
def _make_sc_copy():
    info = plsc.get_sparse_core_info()
    nw = info.num_cores * info.num_subcores
    rows_per_tile = _BATCH // nw
    nc = info.num_cores
    mesh = plsc.VectorSubcoreMesh(core_axis_name="c", subcore_axis_name="s")

    @functools.partial(
        pl.kernel,
        mesh=mesh,
        out_type=jax.ShapeDtypeStruct((_BATCH, _DIM), jnp.float32),
        scratch_types=[pltpu.SemaphoreType.DMA],
    )
    def sc_copy(h_hbm, out_hbm, sem):
        wid = lax.axis_index("s") * nc + lax.axis_index("c")
        base = wid * rows_per_tile
        pltpu.async_copy(
            h_hbm.at[pl.ds(base, rows_per_tile), :],
            out_hbm.at[pl.ds(base, rows_per_tile), :],
            sem,
        ).wait()

    return sc_copy


def kernel(h):
    return _make_sc_copy()(h)


# speedup vs baseline: 36.0243x; 36.0243x over previous
import functools

import jax
import jax.numpy as jnp
from jax import lax
from jax.experimental import pallas as pl
from jax.experimental.pallas import tpu as pltpu
from jax.experimental.pallas import tpu_sc as plsc

_BATCH = 16384
_DIM = 2048

_CHUNK_ROWS = 16


@functools.lru_cache(maxsize=1)
def _make_sc_copy():
    info = plsc.get_sparse_core_info()
    nw = info.num_cores * info.num_subcores
    rows_per_tile = _BATCH // nw
    nc = info.num_cores
    nchunks = rows_per_tile // _CHUNK_ROWS
    mesh = plsc.VectorSubcoreMesh(core_axis_name="c", subcore_axis_name="s")

    @functools.partial(
        pl.kernel,
        mesh=mesh,
        out_type=jax.ShapeDtypeStruct((_BATCH, _DIM), jnp.float32),
        scratch_types=[
            pltpu.VMEM((_CHUNK_ROWS, _DIM), jnp.float32),
            pltpu.VMEM((_CHUNK_ROWS, _DIM), jnp.float32),
            pltpu.SemaphoreType.DMA((2,)),
            pltpu.SemaphoreType.DMA((2,)),
        ],
    )
    def sc_copy(h_hbm, out_hbm, buf0, buf1, rsem, wsem):
        wid = lax.axis_index("s") * nc + lax.axis_index("c")
        base = wid * rows_per_tile
        bufs = (buf0, buf1)

        def rd(i, b):
            return pltpu.make_async_copy(
                h_hbm.at[pl.ds(base + i * _CHUNK_ROWS, _CHUNK_ROWS), :],
                bufs[b], rsem.at[b])

        def wr(i, b):
            return pltpu.make_async_copy(
                bufs[b],
                out_hbm.at[pl.ds(base + i * _CHUNK_ROWS, _CHUNK_ROWS), :],
                wsem.at[b])

        rd(0, 0).start()
        for i in range(nchunks):
            b = i % 2
            if i >= 1:
                wr(i - 1, 1 - b).wait()
            if i + 1 < nchunks:
                rd(i + 1, 1 - b).start()
            rd(i, b).wait()
            wr(i, b).start()
        wr(nchunks - 1, (nchunks - 1) % 2).wait()

    return sc_copy


def kernel(h):
    return _make_sc_copy()(h)


# speedup vs baseline: 36.1006x; 1.0021x over previous
import functools

import jax
import jax.numpy as jnp
from jax import lax
from jax.experimental import pallas as pl
from jax.experimental.pallas import tpu as pltpu
from jax.experimental.pallas import tpu_sc as plsc

_BATCH = 16384
_DIM = 2048

_CHUNK_ROWS = 16


@functools.lru_cache(maxsize=1)
def _make_sc_copy():
    info = plsc.get_sparse_core_info()
    nw = info.num_cores * info.num_subcores
    rows_per_tile = _BATCH // nw
    nc = info.num_cores
    nchunks = rows_per_tile // _CHUNK_ROWS
    mesh = plsc.VectorSubcoreMesh(core_axis_name="c", subcore_axis_name="s")

    @functools.partial(
        pl.kernel,
        mesh=mesh,
        out_type=jax.ShapeDtypeStruct((_BATCH, _DIM), jnp.float32),
        scratch_types=[
            pltpu.VMEM((_CHUNK_ROWS, _DIM), jnp.float32),
            pltpu.VMEM((_CHUNK_ROWS, _DIM), jnp.float32),
            pltpu.VMEM((_CHUNK_ROWS, _DIM), jnp.float32),
            pltpu.SemaphoreType.DMA((3,)),
            pltpu.SemaphoreType.DMA((3,)),
        ],
    )
    def sc_copy(h_hbm, out_hbm, buf0, buf1, buf2, rsem, wsem):
        wid = lax.axis_index("s") * nc + lax.axis_index("c")
        base = wid * rows_per_tile
        bufs = (buf0, buf1, buf2)
        depth = 3

        def rd(i, b):
            return pltpu.make_async_copy(
                h_hbm.at[pl.ds(base + i * _CHUNK_ROWS, _CHUNK_ROWS), :],
                bufs[b], rsem.at[b])

        def wr(i, b):
            return pltpu.make_async_copy(
                bufs[b],
                out_hbm.at[pl.ds(base + i * _CHUNK_ROWS, _CHUNK_ROWS), :],
                wsem.at[b])

        for j in range(depth - 1):
            rd(j, j).start()
        for i in range(nchunks):
            b = i % depth
            nxt = (i + depth - 1) % depth
            if i >= 1:
                wr(i - 1, (i - 1) % depth).wait()
            if i + depth - 1 < nchunks:
                rd(i + depth - 1, nxt).start()
            rd(i, b).wait()
            wr(i, b).start()
        wr(nchunks - 1, (nchunks - 1) % depth).wait()

    return sc_copy


def kernel(h):
    return _make_sc_copy()(h)


# speedup vs baseline: 38.2003x; 1.0582x over previous
import functools

import jax
import jax.numpy as jnp
from jax import lax
from jax.experimental import pallas as pl
from jax.experimental.pallas import tpu as pltpu
from jax.experimental.pallas import tpu_sc as plsc

_BATCH = 16384
_DIM = 2048

_CHUNK_ROWS = 16


@functools.lru_cache(maxsize=1)
def _make_sc_copy():
    info = plsc.get_sparse_core_info()
    nw = info.num_cores * info.num_subcores
    rows_per_tile = _BATCH // nw
    nc = info.num_cores
    nchunks = rows_per_tile // _CHUNK_ROWS
    mesh = plsc.VectorSubcoreMesh(core_axis_name="c", subcore_axis_name="s")

    @functools.partial(
        pl.kernel,
        mesh=mesh,
        out_type=jax.ShapeDtypeStruct((_BATCH, _DIM), jnp.float32),
        scratch_types=[
            pltpu.VMEM_SHARED((16, 3, _CHUNK_ROWS, _DIM), jnp.float32),
            pltpu.SemaphoreType.DMA((3,)),
            pltpu.SemaphoreType.DMA((3,)),
        ],
    )
    def sc_copy(h_hbm, out_hbm, shared, rsem, wsem):
        sid = lax.axis_index("s")
        wid = sid * nc + lax.axis_index("c")
        base = wid * rows_per_tile
        bufs = tuple(shared.at[sid, j] for j in range(3))
        depth = 3

        def rd(i, b):
            return pltpu.make_async_copy(
                h_hbm.at[pl.ds(base + i * _CHUNK_ROWS, _CHUNK_ROWS), :],
                bufs[b], rsem.at[b])

        def wr(i, b):
            return pltpu.make_async_copy(
                bufs[b],
                out_hbm.at[pl.ds(base + i * _CHUNK_ROWS, _CHUNK_ROWS), :],
                wsem.at[b])

        for j in range(depth - 1):
            rd(j, j).start()
        for i in range(nchunks):
            b = i % depth
            nxt = (i + depth - 1) % depth
            if i >= 1:
                wr(i - 1, (i - 1) % depth).wait()
            if i + depth - 1 < nchunks:
                rd(i + depth - 1, nxt).start()
            rd(i, b).wait()
            wr(i, b).start()
        wr(nchunks - 1, (nchunks - 1) % depth).wait()

    return sc_copy


def kernel(h):
    return _make_sc_copy()(h)
